# Initial kernel scaffold; baseline (speedup 1.0000x reference)
import jax
import jax.numpy as jnp
from jax.experimental import pallas as pl


def kernel(x, edge_index, W1, b1, W2, b2, W3, b3):
    raise NotImplementedError("write your pallas kernel here")


# speedup vs baseline: 10.2165x; 10.2165x over previous
import functools

import jax
import jax.numpy as jnp
from jax import lax
from jax.experimental import pallas as pl
from jax.experimental.pallas import tpu as pltpu
from jax.experimental.pallas import tpu_sc as plsc

N = 10000
E = 320000
D = 128
H = 128
C = 40

NP = 10240
W = 48
NC, NS = 2, 16
NW = NC * NS
EPW = E // NW
S = 80
NSTEP = EPW // S
RPT = NP // NS


def _proj_body(x_ref, w1_ref, w2_ref, w3_ref, o_ref):
    wc = jnp.dot(w1_ref[...], jnp.dot(w2_ref[...], w3_ref[...],
                                      preferred_element_type=jnp.float32),
                 preferred_element_type=jnp.float32)
    y = jnp.dot(x_ref[...], wc, preferred_element_type=jnp.float32)
    ones = jnp.ones((NP, 1), jnp.float32)
    zeros = jnp.zeros((NP, W - C - 1), jnp.float32)
    o_ref[...] = jnp.concatenate([y, ones, zeros], axis=1)


_proj = pl.pallas_call(
    _proj_body,
    out_shape=jax.ShapeDtypeStruct((NP, W), jnp.float32),
)


_sc_mesh = plsc.VectorSubcoreMesh(
    core_axis_name="c", subcore_axis_name="s", num_cores=NC, num_subcores=NS)


@functools.partial(
    pl.kernel,
    out_type=jax.ShapeDtypeStruct((NC, NP, W), jnp.float32),
    mesh=_sc_mesh,
    compiler_params=pltpu.CompilerParams(use_tc_tiling_on_sc=False),
    scratch_types=[
        pltpu.VMEM((NSTEP, S), jnp.int32),
        pltpu.VMEM((NSTEP, S), jnp.int32),
        pltpu.VMEM((S, W), jnp.float32),
        pltpu.VMEM_SHARED((NP, W), jnp.float32),
        pltpu.SemaphoreType.DMA,
    ],
)
def _agg(y_hbm, src_hbm, dst_hbm, zero_hbm, out_hbm,
         src_v, dst_v, rows_v, acc_sh, sem):
    c = lax.axis_index("c")
    s = lax.axis_index("s")
    wid = s * NC + c

    pltpu.sync_copy(zero_hbm.at[pl.ds(s * RPT, RPT)],
                    acc_sh.at[pl.ds(s * RPT, RPT)])
    pltpu.sync_copy(src_hbm.at[wid], src_v)
    pltpu.sync_copy(dst_hbm.at[wid], dst_v)
    plsc.subcore_barrier()

    def step(g, carry):
        pltpu.async_copy(y_hbm.at[src_v.at[g]], rows_v, sem).wait()
        pltpu.sync_copy(rows_v, acc_sh.at[dst_v.at[g]], add=True)
        return carry

    lax.fori_loop(0, NSTEP, step, 0)
    plsc.subcore_barrier()
    pltpu.sync_copy(acc_sh.at[pl.ds(s * RPT, RPT)],
                    out_hbm.at[c].at[pl.ds(s * RPT, RPT)])


def _combine_body(p_ref, y_ref, o_ref):
    t = p_ref[0] + p_ref[1] + y_ref[...]
    o_ref[...] = t * (1.0 / t[:, C:C + 1])


_combine = pl.pallas_call(
    _combine_body,
    out_shape=jax.ShapeDtypeStruct((NP, W), jnp.float32),
)


def _final_body(p_ref, y_ref, w2_ref, w3_ref, b1_ref, b2_ref, b3_ref, o_ref):
    t = p_ref[0] + p_ref[1] + y_ref[...]
    y3 = t[:, :C] * (1.0 / t[:, C:C + 1])
    bc = jnp.dot(jnp.dot(b1_ref[...], w2_ref[...],
                         preferred_element_type=jnp.float32) + b2_ref[...],
                 w3_ref[...], preferred_element_type=jnp.float32) + b3_ref[...]
    logits = y3 + bc
    m = jnp.max(logits, axis=1, keepdims=True)
    lse = jnp.log(jnp.sum(jnp.exp(logits - m), axis=1, keepdims=True)) + m
    o_ref[...] = logits - lse


_final = pl.pallas_call(
    _final_body,
    out_shape=jax.ShapeDtypeStruct((NP, C), jnp.float32),
)


def kernel(x, edge_index, W1, b1, W2, b2, W3, b3):
    x_pad = jnp.zeros((NP, D), jnp.float32).at[:N].set(x)
    src = edge_index[0].reshape(NW, NSTEP, S)
    dst = edge_index[1].reshape(NW, NSTEP, S)
    zero = jnp.zeros((NP, W), jnp.float32)
    b1r = b1.reshape(1, H)
    b2r = b2.reshape(1, H)
    b3r = b3.reshape(1, C)

    y = _proj(x_pad, W1, W2, W3)
    parts = _agg(y, src, dst, zero)
    y = _combine(parts, y)
    parts = _agg(y, src, dst, zero)
    y = _combine(parts, y)
    parts = _agg(y, src, dst, zero)
    out = _final(parts, y, W2, W3, b1r, b2r, b3r)
    return out[:N]


# speedup vs baseline: 17.5924x; 1.7220x over previous
import functools

import jax
import jax.numpy as jnp
from jax import lax
from jax.experimental import pallas as pl
from jax.experimental.pallas import tpu as pltpu
from jax.experimental.pallas import tpu_sc as plsc

N = 10000
E = 320000
D = 128
H = 128
C = 40

NP = 10240
W = 48
NC, NS = 2, 16
NW = NC * NS
EPW = E // NW
S = 80
NSTEP = EPW // S
NB = 5
NG = NSTEP // NB
RPT = NP // NS


def _proj_body(x_ref, w1_ref, w2_ref, w3_ref, o_ref):
    wc = jnp.dot(w1_ref[...], jnp.dot(w2_ref[...], w3_ref[...],
                                      preferred_element_type=jnp.float32),
                 preferred_element_type=jnp.float32)
    y = jnp.dot(x_ref[...], wc, preferred_element_type=jnp.float32)
    ones = jnp.ones((NP, 1), jnp.float32)
    zeros = jnp.zeros((NP, W - C - 1), jnp.float32)
    o_ref[...] = jnp.concatenate([y, ones, zeros], axis=1)


_proj = pl.pallas_call(
    _proj_body,
    out_shape=jax.ShapeDtypeStruct((NP, W), jnp.float32),
)


_sc_mesh = plsc.VectorSubcoreMesh(
    core_axis_name="c", subcore_axis_name="s", num_cores=NC, num_subcores=NS)


@functools.partial(
    pl.kernel,
    out_type=jax.ShapeDtypeStruct((NC, NP, W), jnp.float32),
    mesh=_sc_mesh,
    compiler_params=pltpu.CompilerParams(use_tc_tiling_on_sc=False),
    scratch_types=[
        pltpu.VMEM((NSTEP, S), jnp.int32),
        pltpu.VMEM((NSTEP, S), jnp.int32),
        [pltpu.VMEM((S, W), jnp.float32) for _ in range(NB)],
        [pltpu.SemaphoreType.DMA for _ in range(NB)],
        [pltpu.SemaphoreType.DMA for _ in range(NB)],
        pltpu.VMEM_SHARED((NP, W), jnp.float32),
    ],
)
def _agg(y_hbm, src_hbm, dst_hbm, zero_hbm, out_hbm,
         src_v, dst_v, rows_v, gsem, ssem, acc_sh):
    c = lax.axis_index("c")
    s = lax.axis_index("s")
    wid = s * NC + c

    pltpu.sync_copy(zero_hbm.at[pl.ds(s * RPT, RPT)],
                    acc_sh.at[pl.ds(s * RPT, RPT)])
    pltpu.sync_copy(src_hbm.at[wid], src_v)
    pltpu.sync_copy(dst_hbm.at[wid], dst_v)
    plsc.subcore_barrier()

    def group(g, carry):
        gd = [pltpu.async_copy(y_hbm.at[src_v.at[g * NB + b]],
                               rows_v[b], gsem[b]) for b in range(NB)]
        sd = []
        for b in range(NB):
            gd[b].wait()
            sd.append(pltpu.async_copy(rows_v[b],
                                       acc_sh.at[dst_v.at[g * NB + b]],
                                       ssem[b], add=True))
        for b in range(NB):
            sd[b].wait()
        return carry

    lax.fori_loop(0, NG, group, 0)
    plsc.subcore_barrier()
    pltpu.sync_copy(acc_sh.at[pl.ds(s * RPT, RPT)],
                    out_hbm.at[c].at[pl.ds(s * RPT, RPT)])


def _combine_body(p_ref, y_ref, o_ref):
    t = p_ref[0] + p_ref[1] + y_ref[...]
    o_ref[...] = t * (1.0 / t[:, C:C + 1])


_combine = pl.pallas_call(
    _combine_body,
    out_shape=jax.ShapeDtypeStruct((NP, W), jnp.float32),
)


def _final_body(p_ref, y_ref, w2_ref, w3_ref, b1_ref, b2_ref, b3_ref, o_ref):
    t = p_ref[0] + p_ref[1] + y_ref[...]
    y3 = t[:, :C] * (1.0 / t[:, C:C + 1])
    bc = jnp.dot(jnp.dot(b1_ref[...], w2_ref[...],
                         preferred_element_type=jnp.float32) + b2_ref[...],
                 w3_ref[...], preferred_element_type=jnp.float32) + b3_ref[...]
    logits = y3 + bc
    m = jnp.max(logits, axis=1, keepdims=True)
    lse = jnp.log(jnp.sum(jnp.exp(logits - m), axis=1, keepdims=True)) + m
    o_ref[...] = logits - lse


_final = pl.pallas_call(
    _final_body,
    out_shape=jax.ShapeDtypeStruct((NP, C), jnp.float32),
)


def kernel(x, edge_index, W1, b1, W2, b2, W3, b3):
    x_pad = jnp.zeros((NP, D), jnp.float32).at[:N].set(x)
    src = edge_index[0].reshape(NW, NSTEP, S)
    dst = edge_index[1].reshape(NW, NSTEP, S)
    zero = jnp.zeros((NP, W), jnp.float32)
    b1r = b1.reshape(1, H)
    b2r = b2.reshape(1, H)
    b3r = b3.reshape(1, C)

    y = _proj(x_pad, W1, W2, W3)
    parts = _agg(y, src, dst, zero)
    y = _combine(parts, y)
    parts = _agg(y, src, dst, zero)
    y = _combine(parts, y)
    parts = _agg(y, src, dst, zero)
    out = _final(parts, y, W2, W3, b1r, b2r, b3r)
    return out[:N]


# speedup vs baseline: 18.8694x; 1.0726x over previous
import functools

import jax
import jax.numpy as jnp
from jax import lax
from jax.experimental import pallas as pl
from jax.experimental.pallas import tpu as pltpu
from jax.experimental.pallas import tpu_sc as plsc

N = 10000
E = 320000
D = 128
H = 128
C = 40

NP = 10240
W = 48
NC, NS = 2, 16
NW = NC * NS
S = 128
NB = 8
NG = 10
NSTEP = NG * NB
EPW = NSTEP * S
EP = NW * EPW
RPT = NP // NS


def _proj_body(x_ref, w1_ref, w2_ref, w3_ref, o_ref):
    wc = jnp.dot(w1_ref[...], jnp.dot(w2_ref[...], w3_ref[...],
                                      preferred_element_type=jnp.float32),
                 preferred_element_type=jnp.float32)
    y = jnp.dot(x_ref[...], wc, preferred_element_type=jnp.float32)
    ones = jnp.ones((NP, 1), jnp.float32)
    zeros = jnp.zeros((NP, W - C - 1), jnp.float32)
    o_ref[...] = jnp.concatenate([y, ones, zeros], axis=1)


_proj = pl.pallas_call(
    _proj_body,
    out_shape=jax.ShapeDtypeStruct((NP, W), jnp.float32),
)


_sc_mesh = plsc.VectorSubcoreMesh(
    core_axis_name="c", subcore_axis_name="s", num_cores=NC, num_subcores=NS)


@functools.partial(
    pl.kernel,
    out_type=jax.ShapeDtypeStruct((NC, NP, W), jnp.float32),
    mesh=_sc_mesh,
    compiler_params=pltpu.CompilerParams(use_tc_tiling_on_sc=False),
    scratch_types=[
        pltpu.VMEM((NSTEP, S), jnp.int32),
        pltpu.VMEM((NSTEP, S), jnp.int32),
        [pltpu.VMEM((S, W), jnp.float32) for _ in range(NB)],
        [pltpu.SemaphoreType.DMA for _ in range(NB)],
        [pltpu.SemaphoreType.DMA for _ in range(NB)],
        pltpu.VMEM_SHARED((NP, W), jnp.float32),
    ],
)
def _agg(y_hbm, src_hbm, dst_hbm, zero_hbm, out_hbm,
         src_v, dst_v, rows_v, gsem, ssem, acc_sh):
    c = lax.axis_index("c")
    s = lax.axis_index("s")
    wid = s * NC + c

    pltpu.sync_copy(zero_hbm.at[pl.ds(s * RPT, RPT)],
                    acc_sh.at[pl.ds(s * RPT, RPT)])
    pltpu.sync_copy(src_hbm.at[wid], src_v)
    pltpu.sync_copy(dst_hbm.at[wid], dst_v)
    plsc.subcore_barrier()

    def group(g, carry):
        gd = [pltpu.async_copy(y_hbm.at[src_v.at[g * NB + b]],
                               rows_v[b], gsem[b]) for b in range(NB)]
        sd = []
        for b in range(NB):
            gd[b].wait()
            sd.append(pltpu.async_copy(rows_v[b],
                                       acc_sh.at[dst_v.at[g * NB + b]],
                                       ssem[b], add=True))
        for b in range(NB):
            sd[b].wait()
        return carry

    lax.fori_loop(0, NG, group, 0)
    plsc.subcore_barrier()
    pltpu.sync_copy(acc_sh.at[pl.ds(s * RPT, RPT)],
                    out_hbm.at[c].at[pl.ds(s * RPT, RPT)])


def _combine_body(p_ref, y_ref, o_ref):
    t = p_ref[0] + p_ref[1] + y_ref[...]
    o_ref[...] = t * (1.0 / t[:, C:C + 1])


_combine = pl.pallas_call(
    _combine_body,
    out_shape=jax.ShapeDtypeStruct((NP, W), jnp.float32),
)


def _final_body(p_ref, y_ref, w2_ref, w3_ref, b1_ref, b2_ref, b3_ref, o_ref):
    t = p_ref[0] + p_ref[1] + y_ref[...]
    y3 = t[:, :C] * (1.0 / t[:, C:C + 1])
    bc = jnp.dot(jnp.dot(b1_ref[...], w2_ref[...],
                         preferred_element_type=jnp.float32) + b2_ref[...],
                 w3_ref[...], preferred_element_type=jnp.float32) + b3_ref[...]
    logits = y3 + bc
    m = jnp.max(logits, axis=1, keepdims=True)
    lse = jnp.log(jnp.sum(jnp.exp(logits - m), axis=1, keepdims=True)) + m
    o_ref[...] = logits - lse


_final = pl.pallas_call(
    _final_body,
    out_shape=jax.ShapeDtypeStruct((NP, C), jnp.float32),
)


def kernel(x, edge_index, W1, b1, W2, b2, W3, b3):
    x_pad = jnp.zeros((NP, D), jnp.float32).at[:N].set(x)
    pad_idx = (N + jnp.arange(EP - E, dtype=jnp.int32) % (NP - N))
    src = jnp.concatenate([edge_index[0], pad_idx]).reshape(NW, NSTEP, S)
    dst = jnp.concatenate([edge_index[1], pad_idx]).reshape(NW, NSTEP, S)
    zero = jnp.zeros((NP, W), jnp.float32)
    b1r = b1.reshape(1, H)
    b2r = b2.reshape(1, H)
    b3r = b3.reshape(1, C)

    y = _proj(x_pad, W1, W2, W3)
    parts = _agg(y, src, dst, zero)
    y = _combine(parts, y)
    parts = _agg(y, src, dst, zero)
    y = _combine(parts, y)
    parts = _agg(y, src, dst, zero)
    out = _final(parts, y, W2, W3, b1r, b2r, b3r)
    return out[:N]


# speedup vs baseline: 19.7800x; 1.0483x over previous
import functools

import jax
import jax.numpy as jnp
import numpy as np
from jax import lax
from jax.experimental import pallas as pl
from jax.experimental.pallas import tpu as pltpu
from jax.experimental.pallas import tpu_sc as plsc

N = 10000
E = 320000
D = 128
H = 128
C = 40

NP = 10240
W = 48
NC, NS = 2, 16
NW = NC * NS
S = 128
NB = 7
NG = 12
NSTEP = NG * NB
EPW = NSTEP * S
EP = NW * EPW
RPT = NP // NS
CCH = RPT // 2

_APPEND = np.concatenate([
    np.arange(NP, dtype=np.int32),
    N + (np.arange(EP - E - NP, dtype=np.int32) % (NP - N)),
])


def _proj_body(x_ref, w1_ref, w2_ref, w3_ref, o_ref):
    wc = jnp.dot(w1_ref[...], jnp.dot(w2_ref[...], w3_ref[...],
                                      preferred_element_type=jnp.float32),
                 preferred_element_type=jnp.float32)
    y = jnp.dot(x_ref[...], wc, preferred_element_type=jnp.float32)
    ones = jnp.ones((N, 1), jnp.float32)
    zeros = jnp.zeros((N, W - C - 1), jnp.float32)
    o_ref[:N] = jnp.concatenate([y, ones, zeros], axis=1)
    col = lax.broadcasted_iota(jnp.int32, (NP - N, W), 1)
    o_ref[N:] = jnp.where(col == C, 1.0, 0.0)


_proj = pl.pallas_call(
    _proj_body,
    out_shape=jax.ShapeDtypeStruct((NP, W), jnp.float32),
)


_sc_mesh = plsc.VectorSubcoreMesh(
    core_axis_name="c", subcore_axis_name="s", num_cores=NC, num_subcores=NS)


def _make_agg(with_combine):
    out_type = [jax.ShapeDtypeStruct((NC, NP, W), jnp.float32)]
    scratch = [
        pltpu.VMEM((NSTEP, S), jnp.int32),
        pltpu.VMEM((NSTEP, S), jnp.int32),
        [pltpu.VMEM((S, W), jnp.float32) for _ in range(NB)],
        [pltpu.SemaphoreType.DMA for _ in range(NB)],
        [pltpu.SemaphoreType.DMA for _ in range(NB)],
        pltpu.VMEM_SHARED((NP, W), jnp.float32),
    ]
    if with_combine:
        out_type.append(jax.ShapeDtypeStruct((NC, NP, W), jnp.float32))
        scratch += [pltpu.VMEM((CCH, W), jnp.float32),
                    pltpu.VMEM((CCH, W), jnp.float32)]

    def body(*refs):
        if with_combine:
            (pin_hbm, src_hbm, dst_hbm, zero_hbm, out_hbm, y_hbm,
             src_v, dst_v, rows_v, gsem, ssem, acc_sh, ca, cb) = refs
        else:
            (y0_hbm, src_hbm, dst_hbm, zero_hbm, out_hbm,
             src_v, dst_v, rows_v, gsem, ssem, acc_sh) = refs
        c = lax.axis_index("c")
        s = lax.axis_index("s")
        wid = s * NC + c

        pltpu.sync_copy(zero_hbm.at[pl.ds(s * RPT, RPT)],
                        acc_sh.at[pl.ds(s * RPT, RPT)])
        pltpu.sync_copy(src_hbm.at[wid], src_v)
        pltpu.sync_copy(dst_hbm.at[wid], dst_v)

        if with_combine:
            for k in range(2):
                base = s * RPT + k * CCH
                pltpu.sync_copy(pin_hbm.at[0].at[pl.ds(base, CCH)], ca)
                pltpu.sync_copy(pin_hbm.at[1].at[pl.ds(base, CCH)], cb)

                def row(i, carry):
                    t0 = ca[i, pl.ds(0, 16)] + cb[i, pl.ds(0, 16)]
                    t1 = ca[i, pl.ds(16, 16)] + cb[i, pl.ds(16, 16)]
                    t2 = ca[i, pl.ds(32, 16)] + cb[i, pl.ds(32, 16)]
                    inv = (1.0 / t2)[C - 32]
                    ca[i, pl.ds(0, 16)] = t0 * inv
                    ca[i, pl.ds(16, 16)] = t1 * inv
                    ca[i, pl.ds(32, 16)] = t2 * inv
                    return carry

                lax.fori_loop(0, CCH, row, 0)
                pltpu.sync_copy(ca, y_hbm.at[c].at[pl.ds(base, CCH)])
            table = y_hbm.at[c]
        else:
            table = y0_hbm
        plsc.subcore_barrier()

        def group(g, carry):
            gd = [pltpu.async_copy(table.at[src_v.at[g * NB + b]],
                                   rows_v[b], gsem[b]) for b in range(NB)]
            sd = []
            for b in range(NB):
                gd[b].wait()
                sd.append(pltpu.async_copy(rows_v[b],
                                           acc_sh.at[dst_v.at[g * NB + b]],
                                           ssem[b], add=True))
            for b in range(NB):
                sd[b].wait()
            return carry

        lax.fori_loop(0, NG, group, 0)
        plsc.subcore_barrier()
        pltpu.sync_copy(acc_sh.at[pl.ds(s * RPT, RPT)],
                        out_hbm.at[c].at[pl.ds(s * RPT, RPT)])

    return functools.partial(
        pl.kernel,
        out_type=out_type,
        mesh=_sc_mesh,
        compiler_params=pltpu.CompilerParams(use_tc_tiling_on_sc=False),
        scratch_types=scratch,
    )(body)


_agg_first = _make_agg(False)
_agg_next = _make_agg(True)


def _final_body(p_ref, w2_ref, w3_ref, b1_ref, b2_ref, b3_ref, o_ref):
    t = p_ref[0, :N] + p_ref[1, :N]
    y3 = t[:, :C] * (1.0 / t[:, C:C + 1])
    bc = jnp.dot(jnp.dot(b1_ref[...], w2_ref[...],
                         preferred_element_type=jnp.float32) + b2_ref[...],
                 w3_ref[...], preferred_element_type=jnp.float32) + b3_ref[...]
    logits = y3 + bc
    m = jnp.max(logits, axis=1, keepdims=True)
    lse = jnp.log(jnp.sum(jnp.exp(logits - m), axis=1, keepdims=True)) + m
    o_ref[...] = logits - lse


_final = pl.pallas_call(
    _final_body,
    out_shape=jax.ShapeDtypeStruct((N, C), jnp.float32),
)


def kernel(x, edge_index, W1, b1, W2, b2, W3, b3):
    app = jnp.asarray(_APPEND)
    src = jnp.concatenate([edge_index[0], app]).reshape(NW, NSTEP, S)
    dst = jnp.concatenate([edge_index[1], app]).reshape(NW, NSTEP, S)
    zero = jnp.zeros((NP, W), jnp.float32)
    b1r = b1.reshape(1, H)
    b2r = b2.reshape(1, H)
    b3r = b3.reshape(1, C)

    y0 = _proj(x, W1, W2, W3)
    (parts,) = _agg_first(y0, src, dst, zero)
    parts, _ = _agg_next(parts, src, dst, zero)
    parts, _ = _agg_next(parts, src, dst, zero)
    return _final(parts, W2, W3, b1r, b2r, b3r)


# speedup vs baseline: 20.9184x; 1.0576x over previous
import functools

import jax
import jax.numpy as jnp
import numpy as np
from jax import lax
from jax.experimental import pallas as pl
from jax.experimental.pallas import tpu as pltpu
from jax.experimental.pallas import tpu_sc as plsc

N = 10000
E = 320000
D = 128
H = 128
C = 40

NP = 10240
W = 48
NC, NS = 2, 16
NW = NC * NS
S = 128
NB = 7
NG = 12
NSTEP = NG * NB
EPW = NSTEP * S
EP = NW * EPW
RPT = NP // NS
CCH = RPT // 4
NCH = RPT // CCH

_APPEND = np.concatenate([
    np.arange(NP, dtype=np.int32),
    N + (np.arange(EP - E - NP, dtype=np.int32) % (NP - N)),
])


def _proj_body(x_ref, w1_ref, w2_ref, w3_ref, o_ref):
    wc = jnp.dot(w1_ref[...], jnp.dot(w2_ref[...], w3_ref[...],
                                      preferred_element_type=jnp.float32),
                 preferred_element_type=jnp.float32)
    y = jnp.dot(x_ref[...], wc, preferred_element_type=jnp.float32)
    ones = jnp.ones((N, 1), jnp.float32)
    zeros = jnp.zeros((N, W - C - 1), jnp.float32)
    o_ref[:N] = jnp.concatenate([y, ones, zeros], axis=1)
    col = lax.broadcasted_iota(jnp.int32, (NP - N, W), 1)
    o_ref[N:] = jnp.where(col == C, 1.0, 0.0)


_proj = pl.pallas_call(
    _proj_body,
    out_shape=jax.ShapeDtypeStruct((NP, W), jnp.float32),
)


_sc_mesh = plsc.VectorSubcoreMesh(
    core_axis_name="c", subcore_axis_name="s", num_cores=NC, num_subcores=NS)


def _make_agg(with_combine):
    out_type = [jax.ShapeDtypeStruct((NC, NP, W), jnp.float32)]
    scratch = [
        pltpu.VMEM((NSTEP, S), jnp.int32),
        pltpu.VMEM((NSTEP, S), jnp.int32),
        [pltpu.VMEM((S, W), jnp.float32) for _ in range(NB)],
        [pltpu.SemaphoreType.DMA for _ in range(NB)],
        [pltpu.SemaphoreType.DMA for _ in range(NB)],
        pltpu.VMEM_SHARED((NP, W), jnp.float32),
    ]
    if with_combine:
        out_type.append(jax.ShapeDtypeStruct((NC, NP, W), jnp.float32))
        scratch += [[pltpu.VMEM((CCH, W), jnp.float32) for _ in range(2)],
                    [pltpu.VMEM((CCH, W), jnp.float32) for _ in range(2)],
                    [pltpu.SemaphoreType.DMA for _ in range(2)],
                    [pltpu.SemaphoreType.DMA for _ in range(2)]]

    def body(*refs):
        if with_combine:
            (pin_hbm, src_hbm, dst_hbm, zero_hbm, out_hbm, y_hbm,
             src_v, dst_v, rows_v, gsem, ssem, acc_sh,
             ca, cb, lsem, wsem) = refs
        else:
            (y0_hbm, src_hbm, dst_hbm, zero_hbm, out_hbm,
             src_v, dst_v, rows_v, gsem, ssem, acc_sh) = refs
        c = lax.axis_index("c")
        s = lax.axis_index("s")
        wid = s * NC + c

        pltpu.sync_copy(zero_hbm.at[pl.ds(s * RPT, RPT)],
                        acc_sh.at[pl.ds(s * RPT, RPT)])
        pltpu.sync_copy(src_hbm.at[wid], src_v)
        pltpu.sync_copy(dst_hbm.at[wid], dst_v)

        if with_combine:
            def _load(k):
                sl = k % 2
                base = s * RPT + k * CCH
                return (pltpu.async_copy(pin_hbm.at[0].at[pl.ds(base, CCH)],
                                         ca[sl], lsem[sl]),
                        pltpu.async_copy(pin_hbm.at[1].at[pl.ds(base, CCH)],
                                         cb[sl], lsem[sl]))

            def _scale(sl):
                def row4(i, carry):
                    for u in range(4):
                        r = i * 4 + u
                        t0 = ca[sl][r, pl.ds(0, 16)] + cb[sl][r, pl.ds(0, 16)]
                        t1 = ca[sl][r, pl.ds(16, 16)] + cb[sl][r, pl.ds(16, 16)]
                        t2 = ca[sl][r, pl.ds(32, 16)] + cb[sl][r, pl.ds(32, 16)]
                        inv = (1.0 / t2)[C - 32]
                        ca[sl][r, pl.ds(0, 16)] = t0 * inv
                        ca[sl][r, pl.ds(16, 16)] = t1 * inv
                        ca[sl][r, pl.ds(32, 16)] = t2 * inv
                    return carry

                lax.fori_loop(0, CCH // 4, row4, 0)

            lds = [None] * NCH
            sts = [None] * NCH
            lds[0] = _load(0)
            for k in range(NCH):
                sl = k % 2
                if k + 1 < NCH:
                    if k >= 1:
                        sts[k - 1].wait()
                    lds[k + 1] = _load(k + 1)
                lds[k][0].wait()
                lds[k][1].wait()
                _scale(sl)
                base = s * RPT + k * CCH
                sts[k] = pltpu.async_copy(ca[sl],
                                          y_hbm.at[c].at[pl.ds(base, CCH)],
                                          wsem[sl])
            sts[NCH - 2].wait()
            sts[NCH - 1].wait()
            table = y_hbm.at[c]
        else:
            table = y0_hbm
        plsc.subcore_barrier()

        def group(g, carry):
            gd = [pltpu.async_copy(table.at[src_v.at[g * NB + b]],
                                   rows_v[b], gsem[b]) for b in range(NB)]
            sd = []
            for b in range(NB):
                gd[b].wait()
                sd.append(pltpu.async_copy(rows_v[b],
                                           acc_sh.at[dst_v.at[g * NB + b]],
                                           ssem[b], add=True))
            for b in range(NB):
                sd[b].wait()
            return carry

        lax.fori_loop(0, NG, group, 0)
        plsc.subcore_barrier()
        pltpu.sync_copy(acc_sh.at[pl.ds(s * RPT, RPT)],
                        out_hbm.at[c].at[pl.ds(s * RPT, RPT)])

    return functools.partial(
        pl.kernel,
        out_type=out_type,
        mesh=_sc_mesh,
        compiler_params=pltpu.CompilerParams(use_tc_tiling_on_sc=False),
        scratch_types=scratch,
    )(body)


_agg_first = _make_agg(False)
_agg_next = _make_agg(True)


def _final_body(p_ref, w2_ref, w3_ref, b1_ref, b2_ref, b3_ref, o_ref):
    t = p_ref[0, :N] + p_ref[1, :N]
    y3 = t[:, :C] * (1.0 / t[:, C:C + 1])
    bc = jnp.dot(jnp.dot(b1_ref[...], w2_ref[...],
                         preferred_element_type=jnp.float32) + b2_ref[...],
                 w3_ref[...], preferred_element_type=jnp.float32) + b3_ref[...]
    logits = y3 + bc
    m = jnp.max(logits, axis=1, keepdims=True)
    lse = jnp.log(jnp.sum(jnp.exp(logits - m), axis=1, keepdims=True)) + m
    o_ref[...] = logits - lse


_final = pl.pallas_call(
    _final_body,
    out_shape=jax.ShapeDtypeStruct((N, C), jnp.float32),
)


def kernel(x, edge_index, W1, b1, W2, b2, W3, b3):
    app = jnp.asarray(_APPEND)
    src = jnp.concatenate([edge_index[0], app]).reshape(NW, NSTEP, S)
    dst = jnp.concatenate([edge_index[1], app]).reshape(NW, NSTEP, S)
    zero = jnp.zeros((NP, W), jnp.float32)
    b1r = b1.reshape(1, H)
    b2r = b2.reshape(1, H)
    b3r = b3.reshape(1, C)

    y0 = _proj(x, W1, W2, W3)
    (parts,) = _agg_first(y0, src, dst, zero)
    parts, _ = _agg_next(parts, src, dst, zero)
    parts, _ = _agg_next(parts, src, dst, zero)
    return _final(parts, W2, W3, b1r, b2r, b3r)


# speedup vs baseline: 20.9945x; 1.0036x over previous
import functools

import jax
import jax.numpy as jnp
import numpy as np
from jax import lax
from jax.experimental import pallas as pl
from jax.experimental.pallas import tpu as pltpu
from jax.experimental.pallas import tpu_sc as plsc

N = 10000
E = 320000
D = 128
H = 128
C = 40

NP = 10240
W = 48
NC, NS = 2, 16
NW = NC * NS
S = 192
NB = 6
NG = 9
NSTEP = NG * NB
EPW = NSTEP * S
EP = NW * EPW
RPT = NP // NS
CCH = RPT // 8
NCH = RPT // CCH

_APPEND = np.concatenate([
    np.arange(NP, dtype=np.int32),
    N + (np.arange(EP - E - NP, dtype=np.int32) % (NP - N)),
])


def _proj_body(x_ref, w1_ref, w2_ref, w3_ref, o_ref):
    wc = jnp.dot(w1_ref[...], jnp.dot(w2_ref[...], w3_ref[...],
                                      preferred_element_type=jnp.float32),
                 preferred_element_type=jnp.float32)
    y = jnp.dot(x_ref[...], wc, preferred_element_type=jnp.float32)
    ones = jnp.ones((N, 1), jnp.float32)
    zeros = jnp.zeros((N, W - C - 1), jnp.float32)
    o_ref[:N] = jnp.concatenate([y, ones, zeros], axis=1)
    col = lax.broadcasted_iota(jnp.int32, (NP - N, W), 1)
    o_ref[N:] = jnp.where(col == C, 1.0, 0.0)


_proj = pl.pallas_call(
    _proj_body,
    out_shape=jax.ShapeDtypeStruct((NP, W), jnp.float32),
)


_sc_mesh = plsc.VectorSubcoreMesh(
    core_axis_name="c", subcore_axis_name="s", num_cores=NC, num_subcores=NS)


def _make_agg(with_combine):
    out_type = [jax.ShapeDtypeStruct((NC, NP, W), jnp.float32)]
    scratch = [
        pltpu.VMEM((NSTEP, S), jnp.int32),
        pltpu.VMEM((NSTEP, S), jnp.int32),
        [pltpu.VMEM((S, W), jnp.float32) for _ in range(NB)],
        [pltpu.SemaphoreType.DMA for _ in range(NB)],
        [pltpu.SemaphoreType.DMA for _ in range(NB)],
        pltpu.VMEM_SHARED((NP, W), jnp.float32),
    ]
    if with_combine:
        out_type.append(jax.ShapeDtypeStruct((NC, NP, W), jnp.float32))
        scratch += [[pltpu.VMEM((CCH, W), jnp.float32) for _ in range(2)],
                    [pltpu.VMEM((CCH, W), jnp.float32) for _ in range(2)],
                    [pltpu.SemaphoreType.DMA for _ in range(2)],
                    [pltpu.SemaphoreType.DMA for _ in range(2)]]

    def body(*refs):
        if with_combine:
            (pin_hbm, src_hbm, dst_hbm, zero_hbm, out_hbm, y_hbm,
             src_v, dst_v, rows_v, gsem, ssem, acc_sh,
             ca, cb, lsem, wsem) = refs
        else:
            (y0_hbm, src_hbm, dst_hbm, zero_hbm, out_hbm,
             src_v, dst_v, rows_v, gsem, ssem, acc_sh) = refs
        c = lax.axis_index("c")
        s = lax.axis_index("s")
        wid = s * NC + c

        pltpu.sync_copy(zero_hbm.at[pl.ds(s * RPT, RPT)],
                        acc_sh.at[pl.ds(s * RPT, RPT)])
        pltpu.sync_copy(src_hbm.at[wid], src_v)
        pltpu.sync_copy(dst_hbm.at[wid], dst_v)

        if with_combine:
            def _load(k):
                sl = k % 2
                base = s * RPT + k * CCH
                return (pltpu.async_copy(pin_hbm.at[0].at[pl.ds(base, CCH)],
                                         ca[sl], lsem[sl]),
                        pltpu.async_copy(pin_hbm.at[1].at[pl.ds(base, CCH)],
                                         cb[sl], lsem[sl]))

            def _scale(sl):
                def row4(i, carry):
                    for u in range(4):
                        r = i * 4 + u
                        t0 = ca[sl][r, pl.ds(0, 16)] + cb[sl][r, pl.ds(0, 16)]
                        t1 = ca[sl][r, pl.ds(16, 16)] + cb[sl][r, pl.ds(16, 16)]
                        t2 = ca[sl][r, pl.ds(32, 16)] + cb[sl][r, pl.ds(32, 16)]
                        inv = (1.0 / t2)[C - 32]
                        ca[sl][r, pl.ds(0, 16)] = t0 * inv
                        ca[sl][r, pl.ds(16, 16)] = t1 * inv
                        ca[sl][r, pl.ds(32, 16)] = t2 * inv
                    return carry

                lax.fori_loop(0, CCH // 4, row4, 0)

            lds = [None] * NCH
            sts = [None] * NCH
            lds[0] = _load(0)
            for k in range(NCH):
                sl = k % 2
                if k + 1 < NCH:
                    if k >= 1:
                        sts[k - 1].wait()
                    lds[k + 1] = _load(k + 1)
                lds[k][0].wait()
                lds[k][1].wait()
                _scale(sl)
                base = s * RPT + k * CCH
                sts[k] = pltpu.async_copy(ca[sl],
                                          y_hbm.at[c].at[pl.ds(base, CCH)],
                                          wsem[sl])
            sts[NCH - 2].wait()
            sts[NCH - 1].wait()
            table = y_hbm.at[c]
        else:
            table = y0_hbm
        plsc.subcore_barrier()

        def group(g, carry):
            gd = [pltpu.async_copy(table.at[src_v.at[g * NB + b]],
                                   rows_v[b], gsem[b]) for b in range(NB)]
            sd = []
            for b in range(NB):
                gd[b].wait()
                sd.append(pltpu.async_copy(rows_v[b],
                                           acc_sh.at[dst_v.at[g * NB + b]],
                                           ssem[b], add=True))
            for b in range(NB):
                sd[b].wait()
            return carry

        lax.fori_loop(0, NG, group, 0)
        plsc.subcore_barrier()
        pltpu.sync_copy(acc_sh.at[pl.ds(s * RPT, RPT)],
                        out_hbm.at[c].at[pl.ds(s * RPT, RPT)])

    return functools.partial(
        pl.kernel,
        out_type=out_type,
        mesh=_sc_mesh,
        compiler_params=pltpu.CompilerParams(use_tc_tiling_on_sc=False),
        scratch_types=scratch,
    )(body)


_agg_first = _make_agg(False)
_agg_next = _make_agg(True)


def _final_body(p_ref, w2_ref, w3_ref, b1_ref, b2_ref, b3_ref, o_ref):
    t = p_ref[0, :N] + p_ref[1, :N]
    y3 = t[:, :C] * (1.0 / t[:, C:C + 1])
    bc = jnp.dot(jnp.dot(b1_ref[...], w2_ref[...],
                         preferred_element_type=jnp.float32) + b2_ref[...],
                 w3_ref[...], preferred_element_type=jnp.float32) + b3_ref[...]
    logits = y3 + bc
    m = jnp.max(logits, axis=1, keepdims=True)
    lse = jnp.log(jnp.sum(jnp.exp(logits - m), axis=1, keepdims=True)) + m
    o_ref[...] = logits - lse


_final = pl.pallas_call(
    _final_body,
    out_shape=jax.ShapeDtypeStruct((N, C), jnp.float32),
)


def kernel(x, edge_index, W1, b1, W2, b2, W3, b3):
    app = jnp.asarray(_APPEND)
    src = jnp.concatenate([edge_index[0], app]).reshape(NW, NSTEP, S)
    dst = jnp.concatenate([edge_index[1], app]).reshape(NW, NSTEP, S)
    zero = jnp.zeros((NP, W), jnp.float32)
    b1r = b1.reshape(1, H)
    b2r = b2.reshape(1, H)
    b3r = b3.reshape(1, C)

    y0 = _proj(x, W1, W2, W3)
    (parts,) = _agg_first(y0, src, dst, zero)
    parts, _ = _agg_next(parts, src, dst, zero)
    parts, _ = _agg_next(parts, src, dst, zero)
    return _final(parts, W2, W3, b1r, b2r, b3r)


# speedup vs baseline: 22.2075x; 1.0578x over previous
import functools

import jax
import jax.numpy as jnp
import numpy as np
from jax import lax
from jax.experimental import pallas as pl
from jax.experimental.pallas import tpu as pltpu
from jax.experimental.pallas import tpu_sc as plsc

N = 10000
E = 320000
D = 128
H = 128
C = 40

NP = 10240
W = 48
NC, NS = 2, 16
NW = NC * NS
S = 128
NB = 9
NG = 9
NSTEP = NG * NB
EPW = NSTEP * S
EP = NW * EPW
ERW = EPW // S
EREAL = E // S
RPT = NP // NS
CCH = RPT // 8
NCH = RPT // CCH

_APPEND = np.concatenate([
    np.arange(NP, dtype=np.int32),
    N + (np.arange(EP - E - NP, dtype=np.int32) % (NP - N)),
]).reshape(-1, S)
_W30_REAL = EREAL - 30 * ERW
_W30_APP = ERW - _W30_REAL


def _proj_body(x_ref, w1_ref, w2_ref, w3_ref, o_ref):
    wc = jnp.dot(w1_ref[...], jnp.dot(w2_ref[...], w3_ref[...],
                                      preferred_element_type=jnp.float32),
                 preferred_element_type=jnp.float32)
    y = jnp.dot(x_ref[...], wc, preferred_element_type=jnp.float32)
    ones = jnp.ones((N, 1), jnp.float32)
    zeros = jnp.zeros((N, W - C - 1), jnp.float32)
    o_ref[:N] = jnp.concatenate([y, ones, zeros], axis=1)
    col = lax.broadcasted_iota(jnp.int32, (NP - N, W), 1)
    o_ref[N:] = jnp.where(col == C, 1.0, 0.0)


_proj = pl.pallas_call(
    _proj_body,
    out_shape=jax.ShapeDtypeStruct((NP, W), jnp.float32),
)


_sc_mesh = plsc.VectorSubcoreMesh(
    core_axis_name="c", subcore_axis_name="s", num_cores=NC, num_subcores=NS)


def _make_agg(with_combine):
    out_type = [jax.ShapeDtypeStruct((NC, NP, W), jnp.float32)]
    scratch = [
        pltpu.VMEM((ERW, S), jnp.int32),
        pltpu.VMEM((ERW, S), jnp.int32),
        [pltpu.VMEM((S, W), jnp.float32) for _ in range(NB)],
        [pltpu.SemaphoreType.DMA for _ in range(NB)],
        [pltpu.SemaphoreType.DMA for _ in range(NB)],
        pltpu.VMEM_SHARED((NP, W), jnp.float32),
    ]
    if with_combine:
        out_type.append(jax.ShapeDtypeStruct((NC, NP, W), jnp.float32))
        scratch += [[pltpu.VMEM((CCH, W), jnp.float32) for _ in range(2)],
                    [pltpu.VMEM((CCH, W), jnp.float32) for _ in range(2)],
                    [pltpu.SemaphoreType.DMA for _ in range(2)],
                    [pltpu.SemaphoreType.DMA for _ in range(2)]]

    def body(*refs):
        if with_combine:
            (pin_hbm, edges_hbm, app_hbm, zero_hbm, out_hbm, y_hbm,
             src_v, dst_v, rows_v, gsem, ssem, acc_sh,
             ca, cb, lsem, wsem) = refs
        else:
            (y0_hbm, edges_hbm, app_hbm, zero_hbm, out_hbm,
             src_v, dst_v, rows_v, gsem, ssem, acc_sh) = refs
        c = lax.axis_index("c")
        s = lax.axis_index("s")
        wid = s * NC + c

        pltpu.sync_copy(zero_hbm.at[pl.ds(s * RPT, RPT)],
                        acc_sh.at[pl.ds(s * RPT, RPT)])

        @pl.when(wid < 30)
        def _():
            pltpu.sync_copy(edges_hbm.at[0].at[pl.ds(wid * ERW, ERW)], src_v)
            pltpu.sync_copy(edges_hbm.at[1].at[pl.ds(wid * ERW, ERW)], dst_v)

        @pl.when(wid == 30)
        def _():
            for v in (src_v, dst_v):
                pltpu.sync_copy(app_hbm.at[pl.ds(0, _W30_APP)],
                                v.at[pl.ds(_W30_REAL, _W30_APP)])
            pltpu.sync_copy(edges_hbm.at[0].at[pl.ds(30 * ERW, _W30_REAL)],
                            src_v.at[pl.ds(0, _W30_REAL)])
            pltpu.sync_copy(edges_hbm.at[1].at[pl.ds(30 * ERW, _W30_REAL)],
                            dst_v.at[pl.ds(0, _W30_REAL)])

        @pl.when(wid == 31)
        def _():
            pltpu.sync_copy(app_hbm.at[pl.ds(_W30_APP, ERW)], src_v)
            pltpu.sync_copy(app_hbm.at[pl.ds(_W30_APP, ERW)], dst_v)

        if with_combine:
            def _load(k):
                sl = k % 2
                base = s * RPT + k * CCH
                return (pltpu.async_copy(pin_hbm.at[0].at[pl.ds(base, CCH)],
                                         ca[sl], lsem[sl]),
                        pltpu.async_copy(pin_hbm.at[1].at[pl.ds(base, CCH)],
                                         cb[sl], lsem[sl]))

            def _scale(sl):
                def row4(i, carry):
                    for u in range(4):
                        r = i * 4 + u
                        t0 = ca[sl][r, pl.ds(0, 16)] + cb[sl][r, pl.ds(0, 16)]
                        t1 = ca[sl][r, pl.ds(16, 16)] + cb[sl][r, pl.ds(16, 16)]
                        t2 = ca[sl][r, pl.ds(32, 16)] + cb[sl][r, pl.ds(32, 16)]
                        inv = (1.0 / t2)[C - 32]
                        ca[sl][r, pl.ds(0, 16)] = t0 * inv
                        ca[sl][r, pl.ds(16, 16)] = t1 * inv
                        ca[sl][r, pl.ds(32, 16)] = t2 * inv
                    return carry

                lax.fori_loop(0, CCH // 4, row4, 0)

            lds = [None] * NCH
            sts = [None] * NCH
            lds[0] = _load(0)
            for k in range(NCH):
                sl = k % 2
                if k + 1 < NCH:
                    if k >= 1:
                        sts[k - 1].wait()
                    lds[k + 1] = _load(k + 1)
                lds[k][0].wait()
                lds[k][1].wait()
                _scale(sl)
                base = s * RPT + k * CCH
                sts[k] = pltpu.async_copy(ca[sl],
                                          y_hbm.at[c].at[pl.ds(base, CCH)],
                                          wsem[sl])
            sts[NCH - 2].wait()
            sts[NCH - 1].wait()
            table = y_hbm.at[c]
        else:
            table = y0_hbm
        plsc.subcore_barrier()

        def group(g, carry):
            gd = [pltpu.async_copy(table.at[src_v.at[g * NB + b]],
                                   rows_v[b], gsem[b]) for b in range(NB)]
            sd = []
            for b in range(NB):
                gd[b].wait()
                sd.append(pltpu.async_copy(rows_v[b],
                                           acc_sh.at[dst_v.at[g * NB + b]],
                                           ssem[b], add=True))
            for b in range(NB):
                sd[b].wait()
            return carry

        lax.fori_loop(0, NG, group, 0)
        plsc.subcore_barrier()
        pltpu.sync_copy(acc_sh.at[pl.ds(s * RPT, RPT)],
                        out_hbm.at[c].at[pl.ds(s * RPT, RPT)])

    return functools.partial(
        pl.kernel,
        out_type=out_type,
        mesh=_sc_mesh,
        compiler_params=pltpu.CompilerParams(use_tc_tiling_on_sc=False),
        scratch_types=scratch,
    )(body)


_agg_first = _make_agg(False)
_agg_next = _make_agg(True)


def _final_body(p_ref, w2_ref, w3_ref, b1_ref, b2_ref, b3_ref, o_ref):
    t = p_ref[0, :N] + p_ref[1, :N]
    y3 = t[:, :C] * (1.0 / t[:, C:C + 1])
    bc = jnp.dot(jnp.dot(b1_ref[...], w2_ref[...],
                         preferred_element_type=jnp.float32) + b2_ref[...],
                 w3_ref[...], preferred_element_type=jnp.float32) + b3_ref[...]
    logits = y3 + bc
    m = jnp.max(logits, axis=1, keepdims=True)
    lse = jnp.log(jnp.sum(jnp.exp(logits - m), axis=1, keepdims=True)) + m
    o_ref[...] = logits - lse


_final = pl.pallas_call(
    _final_body,
    out_shape=jax.ShapeDtypeStruct((N, C), jnp.float32),
)


def kernel(x, edge_index, W1, b1, W2, b2, W3, b3):
    app = jnp.asarray(_APPEND)
    edges = edge_index.reshape(2, EREAL, S)
    zero = jnp.zeros((NP, W), jnp.float32)
    b1r = b1.reshape(1, H)
    b2r = b2.reshape(1, H)
    b3r = b3.reshape(1, C)

    y0 = _proj(x, W1, W2, W3)
    (parts,) = _agg_first(y0, edges, app, zero)
    parts, _ = _agg_next(parts, edges, app, zero)
    parts, _ = _agg_next(parts, edges, app, zero)
    return _final(parts, W2, W3, b1r, b2r, b3r)


# speedup vs baseline: 22.6279x; 1.0189x over previous
import functools

import jax
import jax.numpy as jnp
import numpy as np
from jax import lax
from jax.experimental import pallas as pl
from jax.experimental.pallas import tpu as pltpu
from jax.experimental.pallas import tpu_sc as plsc

N = 10000
E = 320000
D = 128
H = 128
C = 40

NP = 10240
W = 48
NC, NS = 2, 16
NW = NC * NS
S = 128
NB = 9
NG = 9
NSTEP = NG * NB
EPW = NSTEP * S
EP = NW * EPW
ERW = EPW // S
EREAL = E // S
RPT = NP // NS
CCH = RPT // 8
NCH = RPT // CCH

_APPEND = np.concatenate([
    np.arange(NP, dtype=np.int32),
    N + (np.arange(EP - E - NP, dtype=np.int32) % (NP - N)),
]).reshape(-1, S)
_W30_REAL = EREAL - 30 * ERW
_W30_APP = ERW - _W30_REAL


def _proj_body(x_ref, w1_ref, w2_ref, w3_ref, o_ref):
    wc = jnp.dot(w1_ref[...], jnp.dot(w2_ref[...], w3_ref[...],
                                      preferred_element_type=jnp.float32),
                 preferred_element_type=jnp.float32)
    y = jnp.dot(x_ref[...], wc, preferred_element_type=jnp.float32)
    ones = jnp.ones((N, 1), jnp.float32)
    zeros = jnp.zeros((N, W - C - 1), jnp.float32)
    o_ref[:N] = jnp.concatenate([y, ones, zeros], axis=1)
    col = lax.broadcasted_iota(jnp.int32, (NP - N, W), 1)
    o_ref[N:] = jnp.where(col == C, 1.0, 0.0)


_proj = pl.pallas_call(
    _proj_body,
    out_shape=jax.ShapeDtypeStruct((NP, W), jnp.float32),
)


_sc_mesh = plsc.VectorSubcoreMesh(
    core_axis_name="c", subcore_axis_name="s", num_cores=NC, num_subcores=NS)


def _make_agg(with_combine):
    out_type = [jax.ShapeDtypeStruct((NC, NP, W), jnp.float32)]
    scratch = [
        pltpu.VMEM((ERW, S), jnp.int32),
        pltpu.VMEM((ERW, S), jnp.int32),
        [pltpu.VMEM((S, W), jnp.float32) for _ in range(NB)],
        [pltpu.SemaphoreType.DMA for _ in range(NB)],
        [pltpu.SemaphoreType.DMA for _ in range(NB)],
        pltpu.VMEM_SHARED((NP, W), jnp.float32),
        pltpu.SemaphoreType.DMA,
    ]
    if with_combine:
        out_type.append(jax.ShapeDtypeStruct((NC, NP, W), jnp.float32))
        scratch += [[pltpu.VMEM((CCH, W), jnp.float32) for _ in range(2)],
                    [pltpu.VMEM((CCH, W), jnp.float32) for _ in range(2)],
                    [pltpu.SemaphoreType.DMA for _ in range(2)],
                    [pltpu.SemaphoreType.DMA for _ in range(2)]]

    def body(*refs):
        if with_combine:
            (pin_hbm, edges_hbm, app_hbm, zero_hbm, out_hbm, y_hbm,
             src_v, dst_v, rows_v, gsem, ssem, acc_sh, stg_sem,
             ca, cb, lsem, wsem) = refs
        else:
            (y0_hbm, edges_hbm, app_hbm, zero_hbm, out_hbm,
             src_v, dst_v, rows_v, gsem, ssem, acc_sh, stg_sem) = refs
        c = lax.axis_index("c")
        s = lax.axis_index("s")
        wid = s * NC + c

        pltpu.sync_copy(zero_hbm.at[pl.ds(s * RPT, RPT)],
                        acc_sh.at[pl.ds(s * RPT, RPT)])

        @pl.when(wid < 30)
        def _():
            pltpu.async_copy(edges_hbm.at[0].at[pl.ds(wid * ERW, ERW)],
                             src_v, stg_sem)
            pltpu.async_copy(edges_hbm.at[1].at[pl.ds(wid * ERW, ERW)],
                             dst_v, stg_sem)

        @pl.when(wid == 30)
        def _():
            for v in (src_v, dst_v):
                pltpu.async_copy(app_hbm.at[pl.ds(0, _W30_APP)],
                                 v.at[pl.ds(_W30_REAL, _W30_APP)], stg_sem)
            pltpu.async_copy(edges_hbm.at[0].at[pl.ds(30 * ERW, _W30_REAL)],
                             src_v.at[pl.ds(0, _W30_REAL)], stg_sem)
            pltpu.async_copy(edges_hbm.at[1].at[pl.ds(30 * ERW, _W30_REAL)],
                             dst_v.at[pl.ds(0, _W30_REAL)], stg_sem)

        @pl.when(wid == 31)
        def _():
            pltpu.async_copy(app_hbm.at[pl.ds(_W30_APP, ERW)], src_v, stg_sem)
            pltpu.async_copy(app_hbm.at[pl.ds(_W30_APP, ERW)], dst_v, stg_sem)

        if with_combine:
            def _load(k):
                sl = k % 2
                base = s * RPT + k * CCH
                return (pltpu.async_copy(pin_hbm.at[0].at[pl.ds(base, CCH)],
                                         ca[sl], lsem[sl]),
                        pltpu.async_copy(pin_hbm.at[1].at[pl.ds(base, CCH)],
                                         cb[sl], lsem[sl]))

            def _scale(sl):
                def row4(i, carry):
                    for u in range(8):
                        r = i * 8 + u
                        t0 = ca[sl][r, pl.ds(0, 16)] + cb[sl][r, pl.ds(0, 16)]
                        t1 = ca[sl][r, pl.ds(16, 16)] + cb[sl][r, pl.ds(16, 16)]
                        t2 = ca[sl][r, pl.ds(32, 16)] + cb[sl][r, pl.ds(32, 16)]
                        inv = (1.0 / t2)[C - 32]
                        ca[sl][r, pl.ds(0, 16)] = t0 * inv
                        ca[sl][r, pl.ds(16, 16)] = t1 * inv
                        ca[sl][r, pl.ds(32, 16)] = t2 * inv
                    return carry

                lax.fori_loop(0, CCH // 8, row4, 0)

            lds = [None] * NCH
            sts = [None] * NCH
            lds[0] = _load(0)
            for k in range(NCH):
                sl = k % 2
                if k + 1 < NCH:
                    if k >= 1:
                        sts[k - 1].wait()
                    lds[k + 1] = _load(k + 1)
                lds[k][0].wait()
                lds[k][1].wait()
                _scale(sl)
                base = s * RPT + k * CCH
                sts[k] = pltpu.async_copy(ca[sl],
                                          y_hbm.at[c].at[pl.ds(base, CCH)],
                                          wsem[sl])
            sts[NCH - 2].wait()
            sts[NCH - 1].wait()
            table = y_hbm.at[c]
        else:
            table = y0_hbm
        pltpu.make_async_copy(edges_hbm.at[0].at[pl.ds(0, ERW)],
                              src_v, stg_sem).wait()
        pltpu.make_async_copy(edges_hbm.at[0].at[pl.ds(0, ERW)],
                              dst_v, stg_sem).wait()
        plsc.subcore_barrier()

        def group(g, carry):
            gd = [pltpu.async_copy(table.at[src_v.at[g * NB + b]],
                                   rows_v[b], gsem[b]) for b in range(NB)]
            sd = []
            for b in range(NB):
                gd[b].wait()
                sd.append(pltpu.async_copy(rows_v[b],
                                           acc_sh.at[dst_v.at[g * NB + b]],
                                           ssem[b], add=True))
            for b in range(NB):
                sd[b].wait()
            return carry

        lax.fori_loop(0, NG, group, 0)
        plsc.subcore_barrier()
        pltpu.sync_copy(acc_sh.at[pl.ds(s * RPT, RPT)],
                        out_hbm.at[c].at[pl.ds(s * RPT, RPT)])

    return functools.partial(
        pl.kernel,
        out_type=out_type,
        mesh=_sc_mesh,
        compiler_params=pltpu.CompilerParams(use_tc_tiling_on_sc=False),
        scratch_types=scratch,
    )(body)


_agg_first = _make_agg(False)
_agg_next = _make_agg(True)


def _final_body(p_ref, w2_ref, w3_ref, b1_ref, b2_ref, b3_ref, o_ref):
    t = p_ref[0, :N] + p_ref[1, :N]
    y3 = t[:, :C] * (1.0 / t[:, C:C + 1])
    bc = jnp.dot(jnp.dot(b1_ref[...], w2_ref[...],
                         preferred_element_type=jnp.float32) + b2_ref[...],
                 w3_ref[...], preferred_element_type=jnp.float32) + b3_ref[...]
    logits = y3 + bc
    m = jnp.max(logits, axis=1, keepdims=True)
    lse = jnp.log(jnp.sum(jnp.exp(logits - m), axis=1, keepdims=True)) + m
    o_ref[...] = logits - lse


_final = pl.pallas_call(
    _final_body,
    out_shape=jax.ShapeDtypeStruct((N, C), jnp.float32),
)


def kernel(x, edge_index, W1, b1, W2, b2, W3, b3):
    app = jnp.asarray(_APPEND)
    edges = edge_index.reshape(2, EREAL, S)
    zero = jnp.zeros((NP, W), jnp.float32)
    b1r = b1.reshape(1, H)
    b2r = b2.reshape(1, H)
    b3r = b3.reshape(1, C)

    y0 = _proj(x, W1, W2, W3)
    (parts,) = _agg_first(y0, edges, app, zero)
    parts, _ = _agg_next(parts, edges, app, zero)
    parts, _ = _agg_next(parts, edges, app, zero)
    return _final(parts, W2, W3, b1r, b2r, b3r)
